# Initial kernel scaffold; baseline (speedup 1.0000x reference)
import jax
import jax.numpy as jnp
from jax.experimental import pallas as pl


def kernel(x, pos, seq, ori, batch, edge_index0, edge_index1, edge_index2, edge_index3, features, chem, geo, surf_batch, emb, b0_Wid, b0_Win, b0_Wk1, b0_Wk2, b0_Wc, b0_Wout, b1_Win, b1_Wk1, b1_Wk2, b1_Wc, b1_Wout, b2_Wid, b2_Win, b2_Wk1, b2_Wk2, b2_Wc, b2_Wout, b3_Win, b3_Wk1, b3_Wk2, b3_Wc, b3_Wout, b4_Wid, b4_Win, b4_Wk1, b4_Wk2, b4_Wc, b4_Wout, b5_Win, b5_Wk1, b5_Wk2, b5_Wc, b5_Wout, b6_Wid, b6_Win, b6_Wk1, b6_Wk2, b6_Wc, b6_Wout, b7_Win, b7_Wk1, b7_Wk2, b7_Wc, b7_Wout, mp_W1, mp_b1, mp_W2, mp_b2, kp_W1, kp_W2, cls_W1, cls_b1, cls_W2, cls_b2):
    raise NotImplementedError("write your pallas kernel here")


# speedup vs baseline: 1.9056x; 1.9056x over previous
import functools

import jax
import jax.numpy as jnp
from jax import lax
from jax.experimental import pallas as pl
from jax.experimental.pallas import tpu as pltpu
from jax.experimental.pallas import tpu_sc as plsc

_EPS = 1e-5
_NC = 2
_NS = 16
_NW = _NC * _NS
_K = 128


def _lrelu(x, s):
    return jnp.where(x >= 0, x, s * x)


def _bnact(x, n_true, slope, npad):
    xs = x[:n_true]
    m = xs.mean(axis=0, keepdims=True)
    v = xs.var(axis=0, keepdims=True)
    y = _lrelu((xs - m) / jnp.sqrt(v + _EPS), slope)
    return jnp.pad(y, ((0, npad - n_true), (0, 0)))


def _mm(x, w, blk, add=None):
    npad, ci = x.shape
    co = w.shape[1]
    args = [x, w] + ([add] if add is not None else [])

    def body(*refs):
        if add is not None:
            x_ref, w_ref, a_ref, o_ref = refs
        else:
            x_ref, w_ref, o_ref = refs
        y = jnp.dot(x_ref[...], w_ref[...], preferred_element_type=jnp.float32)
        if add is not None:
            y = y + a_ref[...]
        o_ref[...] = y

    in_specs = [pl.BlockSpec((blk, ci), lambda i: (i, 0)),
                pl.BlockSpec((ci, co), lambda i: (0, 0))]
    if add is not None:
        in_specs.append(pl.BlockSpec((blk, co), lambda i: (i, 0)))
    return pl.pallas_call(
        body, grid=(npad // blk,), in_specs=in_specs,
        out_specs=pl.BlockSpec((blk, co), lambda i: (i, 0)),
        out_shape=jax.ShapeDtypeStruct((npad, co), jnp.float32))(*args)


def _mm_sum2(p0, p1, w, blk):
    npad, wi = p0.shape
    co = w.shape[1]

    def body(a_ref, b_ref, w_ref, o_ref):
        o_ref[...] = jnp.dot(a_ref[...] + b_ref[...], w_ref[...],
                             preferred_element_type=jnp.float32)

    return pl.pallas_call(
        body, grid=(npad // blk,),
        in_specs=[pl.BlockSpec((blk, wi), lambda i: (i, 0)),
                  pl.BlockSpec((blk, wi), lambda i: (i, 0)),
                  pl.BlockSpec((wi, co), lambda i: (0, 0))],
        out_specs=pl.BlockSpec((blk, co), lambda i: (i, 0)),
        out_shape=jax.ShapeDtypeStruct((npad, co), jnp.float32))(p0, p1, w)


def _pool(x, blk):
    npad, c = x.shape
    x3 = x.reshape(npad // 2, 2, c)

    def body(x_ref, o_ref):
        xb = x_ref[...]
        o_ref[...] = 0.5 * (xb[:, 0, :] + xb[:, 1, :])

    return pl.pallas_call(
        body, grid=(npad // 2 // blk,),
        in_specs=[pl.BlockSpec((blk, 2, c), lambda i: (i, 0, 0))],
        out_specs=pl.BlockSpec((blk, c), lambda i: (i, 0)),
        out_shape=jax.ShapeDtypeStruct((npad // 2, c), jnp.float32))(x3)


def _embed(xf, emb, blk):
    npad = xf.shape[0]
    ne, ce = emb.shape

    def body(x_ref, e_ref, o_ref):
        oh = (x_ref[...] == lax.broadcasted_iota(
            jnp.int32, (blk, ne), 1).astype(jnp.float32)).astype(jnp.float32)
        o_ref[...] = jnp.dot(oh, e_ref[...], preferred_element_type=jnp.float32,
                             precision=lax.Precision.HIGHEST)

    return pl.pallas_call(
        body, grid=(npad // blk,),
        in_specs=[pl.BlockSpec((blk, 1), lambda i: (i, 0)),
                  pl.BlockSpec((ne, ce), lambda i: (0, 0))],
        out_specs=pl.BlockSpec((blk, ce), lambda i: (i, 0)),
        out_shape=jax.ShapeDtypeStruct((npad, ce), jnp.float32))(xf, emb)


def _gatemsg(rel, hs, wk1, wk2, e_true, blk):
    ep = rel.shape[0]
    w = wk2.shape[1]

    def body(r_ref, h_ref, k1_ref, k2_ref, o_ref):
        i = pl.program_id(0)
        rb = r_ref[...]
        k1 = k1_ref[...]
        t = jnp.zeros((blk, k1.shape[1]), jnp.float32)
        for c in range(3):
            t = t + rb[:, c:c + 1] * k1[c:c + 1, :]
        t = _lrelu(t, 0.2)
        g = jax.nn.sigmoid(jnp.dot(t, k2_ref[...],
                                   preferred_element_type=jnp.float32))
        row = lax.broadcasted_iota(jnp.int32, (blk, 1), 0) + i * blk
        g = g * (row < e_true).astype(jnp.float32)
        o_ref[...] = h_ref[...] * g

    return pl.pallas_call(
        body, grid=(ep // blk,),
        in_specs=[pl.BlockSpec((blk, 16), lambda i: (i, 0)),
                  pl.BlockSpec((blk, w), lambda i: (i, 0)),
                  pl.BlockSpec((16, wk1.shape[1]), lambda i: (0, 0)),
                  pl.BlockSpec((wk1.shape[1], w), lambda i: (0, 0))],
        out_specs=pl.BlockSpec((blk, w), lambda i: (i, 0)),
        out_shape=jax.ShapeDtypeStruct((ep, w), jnp.float32))(rel, hs, wk1, wk2)


def _chem_mm(x3, w, b, blk):
    npad, _, k = x3.shape
    cc = w.shape[1]

    def body(c_ref, w_ref, b_ref, o_ref):
        cb = c_ref[...]
        for j in range(8):
            o_ref[:, j, :] = jnp.dot(
                cb[:, j, :], w_ref[...],
                preferred_element_type=jnp.float32) + b_ref[...]

    return pl.pallas_call(
        body, grid=(npad // blk,),
        in_specs=[pl.BlockSpec((blk, 8, k), lambda i: (i, 0, 0)),
                  pl.BlockSpec((k, cc), lambda i: (0, 0)),
                  pl.BlockSpec((1, cc), lambda i: (0, 0))],
        out_specs=pl.BlockSpec((blk, 8, cc), lambda i: (i, 0, 0)),
        out_shape=jax.ShapeDtypeStruct((npad, 8, cc), jnp.float32))(x3, w, b)


def _segsum_mm(x, segf, w1, slope, blk):
    npad, ci = x.shape
    co = w1.shape[1]

    def body(x_ref, s_ref, w_ref, on_ref, oc_ref):
        i = pl.program_id(0)

        @pl.when(i == 0)
        def _():
            on_ref[...] = jnp.zeros_like(on_ref)
            oc_ref[...] = jnp.zeros_like(oc_ref)

        g = _lrelu(jnp.dot(x_ref[...], w_ref[...],
                           preferred_element_type=jnp.float32), slope)
        oh = (s_ref[...] == lax.broadcasted_iota(
            jnp.int32, (blk, 8), 1).astype(jnp.float32)).astype(jnp.float32)
        on_ref[...] += lax.dot_general(
            oh, g, (((0,), (0,)), ((), ())),
            preferred_element_type=jnp.float32,
            precision=lax.Precision.HIGHEST)
        oc_ref[...] += lax.dot_general(
            oh, jnp.ones((blk, 1), jnp.float32), (((0,), (0,)), ((), ())),
            preferred_element_type=jnp.float32,
            precision=lax.Precision.HIGHEST)

    return pl.pallas_call(
        body, grid=(npad // blk,),
        in_specs=[pl.BlockSpec((blk, ci), lambda i: (i, 0)),
                  pl.BlockSpec((blk, 1), lambda i: (i, 0)),
                  pl.BlockSpec((ci, co), lambda i: (0, 0))],
        out_specs=[pl.BlockSpec((8, co), lambda i: (0, 0)),
                   pl.BlockSpec((8, 1), lambda i: (0, 0))],
        out_shape=[jax.ShapeDtypeStruct((8, co), jnp.float32),
                   jax.ShapeDtypeStruct((8, 1), jnp.float32)])(x, segf, w1)


def _segsum(x, segf, blk):
    npad, ci = x.shape

    def body(x_ref, s_ref, on_ref, oc_ref):
        i = pl.program_id(0)

        @pl.when(i == 0)
        def _():
            on_ref[...] = jnp.zeros_like(on_ref)
            oc_ref[...] = jnp.zeros_like(oc_ref)

        oh = (s_ref[...] == lax.broadcasted_iota(
            jnp.int32, (blk, 8), 1).astype(jnp.float32)).astype(jnp.float32)
        on_ref[...] += lax.dot_general(
            oh, x_ref[...], (((0,), (0,)), ((), ())),
            preferred_element_type=jnp.float32,
            precision=lax.Precision.HIGHEST)
        oc_ref[...] += lax.dot_general(
            oh, jnp.ones((blk, 1), jnp.float32), (((0,), (0,)), ((), ())),
            preferred_element_type=jnp.float32,
            precision=lax.Precision.HIGHEST)

    return pl.pallas_call(
        body, grid=(npad // blk,),
        in_specs=[pl.BlockSpec((blk, ci), lambda i: (i, 0)),
                  pl.BlockSpec((blk, 1), lambda i: (i, 0))],
        out_specs=[pl.BlockSpec((8, ci), lambda i: (0, 0)),
                   pl.BlockSpec((8, 1), lambda i: (0, 0))],
        out_shape=[jax.ShapeDtypeStruct((8, ci), jnp.float32),
                   jax.ShapeDtypeStruct((8, 1), jnp.float32)])(x, segf)


def _cls_concat(segnum, cnt_s, num_st, cnt_st, kp_w2):
    cs = kp_w2.shape[1]
    ct = num_st.shape[1]

    def body(sn_ref, cs_ref, nst_ref, cst_ref, w2_ref, o_ref):
        osurf = jnp.dot(sn_ref[...], w2_ref[...],
                        preferred_element_type=jnp.float32,
                        precision=lax.Precision.HIGHEST)
        o_ref[:, 0:cs] = osurf / jnp.maximum(cs_ref[...], 1.0)
        o_ref[:, cs:cs + ct] = nst_ref[...] / jnp.maximum(cst_ref[...], 1.0)

    return pl.pallas_call(
        body, grid=(1,),
        in_specs=[pl.BlockSpec(segnum.shape, lambda i: (0, 0)),
                  pl.BlockSpec((8, 1), lambda i: (0, 0)),
                  pl.BlockSpec(num_st.shape, lambda i: (0, 0)),
                  pl.BlockSpec((8, 1), lambda i: (0, 0)),
                  pl.BlockSpec(kp_w2.shape, lambda i: (0, 0))],
        out_specs=pl.BlockSpec((8, cs + ct), lambda i: (0, 0)),
        out_shape=jax.ShapeDtypeStruct((8, cs + ct), jnp.float32))(
            segnum, cnt_s, num_st, cnt_st, kp_w2)


def _cls_layer(o, w, b, blk):
    n8, ci = o.shape
    co = w.shape[1]

    def body(o_ref, w_ref, b_ref, h_ref):
        k = pl.program_id(0)

        @pl.when(k == 0)
        def _():
            h_ref[...] = jnp.broadcast_to(b_ref[...], (n8, co))

        ob = o_ref[...]
        m = jnp.mean(ob, axis=0, keepdims=True)
        v = jnp.mean((ob - m) * (ob - m), axis=0, keepdims=True)
        on = _lrelu((ob - m) * lax.rsqrt(v + _EPS), 0.2)
        h_ref[...] += jnp.dot(on, w_ref[...],
                              preferred_element_type=jnp.float32)

    return pl.pallas_call(
        body, grid=(ci // blk,),
        in_specs=[pl.BlockSpec((n8, blk), lambda k: (0, k)),
                  pl.BlockSpec((blk, co), lambda k: (k, 0)),
                  pl.BlockSpec((1, co), lambda k: (0, 0))],
        out_specs=pl.BlockSpec((n8, co), lambda k: (0, 0)),
        out_shape=jax.ShapeDtypeStruct((n8, co), jnp.float32))(o, w, b)


_MESH = dict(core_axis_name="c", subcore_axis_name="s")
_SC_PARAMS = pltpu.CompilerParams(use_tc_tiling_on_sc=False)


def _sc_rel(srcp, dstp, pos16):
    ep = srcp.shape[0]
    per_w = ep // _NW
    cpw = per_w // _K

    @functools.partial(
        pl.kernel, mesh=plsc.VectorSubcoreMesh(**_MESH),
        compiler_params=_SC_PARAMS,
        out_type=jax.ShapeDtypeStruct((ep, 16), jnp.float32),
        scratch_types=[
            pltpu.VMEM((_K,), jnp.int32),
            pltpu.VMEM((_K,), jnp.int32),
            pltpu.VMEM((_K, 16), jnp.float32),
            pltpu.VMEM((_K, 16), jnp.float32),
            pltpu.SemaphoreType.DMA,
            pltpu.SemaphoreType.DMA,
        ])
    def k(src_h, dst_h, pos_h, rel_h, sidx, didx, srows, drows, sem1, sem2):
        wid = lax.axis_index("s") * _NC + lax.axis_index("c")
        base_w = wid * per_w

        def chunk(j, carry):
            base = base_w + j * _K
            pltpu.sync_copy(src_h.at[pl.ds(base, _K)], sidx)
            pltpu.sync_copy(dst_h.at[pl.ds(base, _K)], didx)
            cp1 = pltpu.async_copy(pos_h.at[sidx], srows, sem1)
            cp2 = pltpu.async_copy(pos_h.at[didx], drows, sem2)
            cp1.wait()
            cp2.wait()

            def row(i, c2):
                srows[i, :] = srows[i, :] - drows[i, :]
                return c2

            lax.fori_loop(0, _K, row, 0, unroll=8)
            pltpu.sync_copy(srows, rel_h.at[pl.ds(base, _K), :])
            return carry

        lax.fori_loop(0, cpw, chunk, 0)

    return k(srcp, dstp, pos16)


def _sc_gather(table, idx):
    npad, w = table.shape
    ep = idx.shape[0]
    per_w = ep // _NW
    cpw = per_w // _K

    @functools.partial(
        pl.kernel, mesh=plsc.VectorSubcoreMesh(**_MESH),
        compiler_params=_SC_PARAMS,
        out_type=jax.ShapeDtypeStruct((ep, w), jnp.float32),
        scratch_types=[
            pltpu.VMEM((_K,), jnp.int32),
            pltpu.VMEM((_K, w), jnp.float32),
            pltpu.SemaphoreType.DMA,
        ])
    def k(t_h, i_h, o_h, iv, rows, sem):
        wid = lax.axis_index("s") * _NC + lax.axis_index("c")
        base_w = wid * per_w

        def chunk(j, carry):
            base = base_w + j * _K
            pltpu.sync_copy(i_h.at[pl.ds(base, _K)], iv)
            pltpu.async_copy(t_h.at[iv], rows, sem).wait()
            pltpu.sync_copy(rows, o_h.at[pl.ds(base, _K), :])
            return carry

        lax.fori_loop(0, cpw, chunk, 0)

    return k(table, idx)


def _sc_scatter(msg, dstp, npad, zrows):
    ep, w = msg.shape
    per_w = ep // _NW
    cpw = per_w // _K
    npw_sub = npad // _NS
    zr = zrows.shape[0]
    zc = npw_sub // zr

    @functools.partial(
        pl.kernel, mesh=plsc.VectorSubcoreMesh(**_MESH),
        compiler_params=_SC_PARAMS,
        out_type=jax.ShapeDtypeStruct((_NC, npad, w), jnp.float32),
        scratch_types=[
            pltpu.VMEM((_K,), jnp.int32),
            pltpu.VMEM((_K, w), jnp.float32),
            pltpu.VMEM_SHARED((npad, w), jnp.float32),
        ])
    def k(m_h, d_h, z_h, o_h, div, rows, agg):
        cid = lax.axis_index("c")
        sid = lax.axis_index("s")
        wid = sid * _NC + cid

        def zchunk(j, carry):
            pltpu.sync_copy(z_h, agg.at[pl.ds(sid * npw_sub + j * zr, zr), :])
            return carry

        lax.fori_loop(0, zc, zchunk, 0)
        plsc.subcore_barrier()

        def chunk(j, carry):
            base = wid * per_w + j * _K
            pltpu.sync_copy(d_h.at[pl.ds(base, _K)], div)
            pltpu.sync_copy(m_h.at[pl.ds(base, _K), :], rows)
            pltpu.sync_copy(rows, agg.at[div], add=True)
            return carry

        lax.fori_loop(0, cpw, chunk, 0)
        plsc.subcore_barrier()
        pltpu.sync_copy(agg.at[pl.ds(sid * npw_sub, npw_sub), :],
                        o_h.at[cid, pl.ds(sid * npw_sub, npw_sub), :])

    return k(msg, dstp, zrows)


def _round_up(a, b):
    return (a + b - 1) // b * b


def kernel(x, pos, seq, ori, batch, edge_index0, edge_index1, edge_index2,
           edge_index3, features, chem, geo, surf_batch, emb,
           b0_Wid, b0_Win, b0_Wk1, b0_Wk2, b0_Wc, b0_Wout,
           b1_Win, b1_Wk1, b1_Wk2, b1_Wc, b1_Wout,
           b2_Wid, b2_Win, b2_Wk1, b2_Wk2, b2_Wc, b2_Wout,
           b3_Win, b3_Wk1, b3_Wk2, b3_Wc, b3_Wout,
           b4_Wid, b4_Win, b4_Wk1, b4_Wk2, b4_Wc, b4_Wout,
           b5_Win, b5_Wk1, b5_Wk2, b5_Wc, b5_Wout,
           b6_Wid, b6_Win, b6_Wk1, b6_Wk2, b6_Wc, b6_Wout,
           b7_Win, b7_Wk1, b7_Wk2, b7_Wc, b7_Wout,
           mp_W1, mp_b1, mp_W2, mp_b2, kp_W1, kp_W2,
           cls_W1, cls_b1, cls_W2, cls_b2):
    f32 = jnp.float32
    n0 = x.shape[0]
    ln = [n0, n0 // 2, n0 // 4, n0 // 8]
    np0 = _round_up(n0, 1024)
    npl = [np0, np0 // 2, np0 // 4, np0 // 8]
    blkl = [512, 512, 256, 128]

    edges = [edge_index0, edge_index1, edge_index2, edge_index3]
    srcp, dstp, etrue = [], [], []
    for e in edges:
        ecount = e.shape[1]
        ep = _round_up(ecount, _NW * _K)
        srcp.append(jnp.pad(e[0].astype(jnp.int32), (0, ep - ecount)))
        dstp.append(jnp.pad(e[1].astype(jnp.int32), (0, ep - ecount)))
        etrue.append(ecount)

    params = [
        dict(Wid=b0_Wid, Win=b0_Win, Wk1=b0_Wk1, Wk2=b0_Wk2, Wc=b0_Wc, Wout=b0_Wout),
        dict(Wid=None, Win=b1_Win, Wk1=b1_Wk1, Wk2=b1_Wk2, Wc=b1_Wc, Wout=b1_Wout),
        dict(Wid=b2_Wid, Win=b2_Win, Wk1=b2_Wk1, Wk2=b2_Wk2, Wc=b2_Wc, Wout=b2_Wout),
        dict(Wid=None, Win=b3_Win, Wk1=b3_Wk1, Wk2=b3_Wk2, Wc=b3_Wc, Wout=b3_Wout),
        dict(Wid=b4_Wid, Win=b4_Win, Wk1=b4_Wk1, Wk2=b4_Wk2, Wc=b4_Wc, Wout=b4_Wout),
        dict(Wid=None, Win=b5_Win, Wk1=b5_Wk1, Wk2=b5_Wk2, Wc=b5_Wc, Wout=b5_Wout),
        dict(Wid=b6_Wid, Win=b6_Win, Wk1=b6_Wk1, Wk2=b6_Wk2, Wc=b6_Wc, Wout=b6_Wout),
        dict(Wid=None, Win=b7_Win, Wk1=b7_Wk1, Wk2=b7_Wk2, Wc=b7_Wc, Wout=b7_Wout),
    ]

    xf = jnp.pad(x.astype(f32)[:, None], ((0, np0 - n0), (0, 0)))
    pos16 = jnp.pad(pos, ((0, np0 - n0), (0, 13)))
    struct_f = jnp.pad(batch.astype(f32)[::8][:, None],
                       ((0, npl[3] - ln[3]), (0, 0)), constant_values=8.0)
    surf_f = jnp.pad(surf_batch.astype(f32)[:, None],
                     ((0, np0 - n0), (0, 0)), constant_values=8.0)

    xcur = _embed(xf, emb, blkl[0])
    rel16 = None
    num_st = cnt_st = None
    for i in range(8):
        lev = i // 2
        blk = blkl[lev]
        p = params[i]
        w = p['Wc'].shape[0]
        if i % 2 == 0:
            rel16 = _sc_rel(srcp[lev], dstp[lev], pos16)
        h = _mm(_bnact(xcur, ln[lev], 0.2, npl[lev]), p['Win'], blk)
        if p['Wid'] is not None:
            idn = _mm(_bnact(xcur, ln[lev], 0.1, npl[lev]), p['Wid'], blk)
        else:
            idn = xcur
        h2 = _bnact(h, ln[lev], 0.2, npl[lev])
        hs = _sc_gather(h2, srcp[lev])
        wk1p = jnp.pad(p['Wk1'], ((0, 13), (0, 0)))
        msg = _gatemsg(rel16, hs, wk1p, p['Wk2'], etrue[lev], 1024)
        zrows = jnp.zeros((npl[lev] // _NS // max(1, npl[lev] // _NS // 392),
                           w), f32)
        parts = _sc_scatter(msg, dstp[lev], npl[lev], zrows)
        h3 = _mm_sum2(parts[0], parts[1], p['Wc'], blk)
        xcur = _mm(_bnact(h3, ln[lev], 0.1, npl[lev]), p['Wout'], blk, add=idn)
        if i == 7:
            num_st, cnt_st = _segsum(xcur, struct_f, blkl[3])
        elif i % 2 == 1:
            xcur = _pool(xcur, blkl[lev + 1])
            pos16 = _pool(pos16, blkl[lev + 1])

    chemp = jnp.pad(chem, ((0, np0 - n0), (0, 0), (0, 0)))
    c1 = _chem_mm(chemp, mp_W1, mp_b1[None, :], blkl[0])
    cs = c1[:n0]
    cm = cs.mean(axis=(0, 1), keepdims=True)
    cv = cs.var(axis=(0, 1), keepdims=True)
    cn = _lrelu((cs - cm) / jnp.sqrt(cv + _EPS), 0.1)
    cn = jnp.pad(cn, ((0, np0 - n0), (0, 0), (0, 0)))
    c2 = _chem_mm(cn, mp_W2, mp_b2[None, :], blkl[0])
    cmax = c2.max(axis=-2)
    feat = jnp.concatenate([
        jnp.pad(geo, ((0, np0 - n0), (0, 0))),
        cmax,
        jnp.pad(features, ((0, np0 - n0), (0, 0)))], axis=1)
    segnum, cnt_s = _segsum_mm(feat, surf_f, kp_W1, 0.1, blkl[0])

    o = _cls_concat(segnum, cnt_s, num_st, cnt_st, kp_W2)
    h1 = _cls_layer(o, cls_W1, cls_b1[None, :], 512)
    return _cls_layer(h1, cls_W2, cls_b2[None, :], 512)
